# Initial kernel scaffold; baseline (speedup 1.0000x reference)
import jax
import jax.numpy as jnp
from jax.experimental import pallas as pl


def kernel(idx, wte, wpe, reasoning_prompts):
    raise NotImplementedError("write your pallas kernel here")


# speedup vs baseline: 1.1541x; 1.1541x over previous
import functools

import jax
import jax.numpy as jnp
from jax import lax
from jax.experimental import pallas as pl
from jax.experimental.pallas import tpu as pltpu
from jax.experimental.pallas import tpu_sc as plsc

_B = 4
_T = 2048
_D = 128
_NPROMPT = 8

_INFO = plsc.get_sparse_core_info()
_NC = _INFO.num_cores
_NS = _INFO.num_subcores
_NW = _NC * _NS
_TPW = _T // _NW
_LANES = 16
_CHUNKS = _D // _LANES


def _body(idx_hbm, wte_hbm, wpe_hbm, prm_hbm, out_hbm,
          idx_v, rows_v, wpe_v, prm_v, sem):
    wid = lax.axis_index("s") * _NC + lax.axis_index("c")
    t0 = wid * _TPW

    for b in range(_B):
        pltpu.sync_copy(idx_hbm.at[b, pl.ds(t0, _TPW)], idx_v.at[b])
    pltpu.sync_copy(wpe_hbm.at[pl.ds(t0, _TPW)], wpe_v)

    @pl.when(wid < _B)
    def _():
        pltpu.sync_copy(prm_hbm, prm_v)
        pltpu.sync_copy(prm_v, out_hbm.at[wid, pl.ds(0, _NPROMPT)])

    for b in range(_B):
        pltpu.async_copy(wte_hbm.at[idx_v.at[b]], rows_v, sem).wait()

        def _add_row(i, _):
            for j in range(_CHUNKS):
                sl = pl.ds(j * _LANES, _LANES)
                rows_v[i, sl] = rows_v[i, sl] + wpe_v[i, sl]
            return _

        lax.fori_loop(0, _TPW, _add_row, None)
        pltpu.sync_copy(rows_v, out_hbm.at[b, pl.ds(_NPROMPT + t0, _TPW)])


@functools.partial(jax.jit, static_argnames=())
def kernel(idx, wte, wpe, reasoning_prompts):
    b, t = idx.shape
    assert (b, t) == (_B, _T)
    mesh = plsc.VectorSubcoreMesh(core_axis_name="c", subcore_axis_name="s")
    run = pl.kernel(
        _body,
        out_type=jax.ShapeDtypeStruct((_B, _NPROMPT + _T, _D), jnp.float32),
        mesh=mesh,
        scratch_types=[
            pltpu.VMEM((_B, _TPW), jnp.int32),
            pltpu.VMEM((_TPW, _D), jnp.float32),
            pltpu.VMEM((_TPW, _D), jnp.float32),
            pltpu.VMEM((_NPROMPT, _D), jnp.float32),
            pltpu.SemaphoreType.DMA,
        ],
    )
    return run(idx.astype(jnp.int32), wte, wpe, reasoning_prompts)


# speedup vs baseline: 1.2375x; 1.0722x over previous
import functools

import jax
import jax.numpy as jnp
from jax import lax
from jax.experimental import pallas as pl
from jax.experimental.pallas import tpu as pltpu
from jax.experimental.pallas import tpu_sc as plsc

_B = 4
_T = 2048
_D = 128
_NPROMPT = 8

_INFO = plsc.get_sparse_core_info()
_NC = _INFO.num_cores
_NS = _INFO.num_subcores
_NW = _NC * _NS
_TPW = _T // _NW
_LANES = 16
_CHUNKS = _D // _LANES


def _body(idx_hbm, wte_hbm, wpe_hbm, prm_hbm, out_hbm,
          idx_v, rows_v, prm_v, sem0, sem1):
    sems = (sem0, sem1)
    wid = lax.axis_index("s") * _NC + lax.axis_index("c")
    t0 = wid * _TPW

    for b in range(_B):
        pltpu.sync_copy(idx_hbm.at[b, pl.ds(t0, _TPW)], idx_v.at[b])

    @pl.when(wid < _B)
    def _():
        pltpu.sync_copy(prm_hbm, prm_v)
        pltpu.sync_copy(prm_v, out_hbm.at[wid, pl.ds(0, _NPROMPT)])

    def _start(b):
        buf = rows_v.at[b % 2]
        pltpu.sync_copy(wpe_hbm.at[pl.ds(t0, _TPW)], buf)
        return pltpu.async_copy(wte_hbm.at[idx_v.at[b]], buf,
                                sems[b % 2], add=True)

    cp = _start(0)
    for b in range(_B):
        nxt = _start(b + 1) if b + 1 < _B else None
        cp.wait()
        pltpu.sync_copy(rows_v.at[b % 2],
                        out_hbm.at[b, pl.ds(_NPROMPT + t0, _TPW)])
        cp = nxt


@functools.partial(jax.jit, static_argnames=())
def kernel(idx, wte, wpe, reasoning_prompts):
    b, t = idx.shape
    assert (b, t) == (_B, _T)
    mesh = plsc.VectorSubcoreMesh(core_axis_name="c", subcore_axis_name="s")
    run = pl.kernel(
        _body,
        out_type=jax.ShapeDtypeStruct((_B, _NPROMPT + _T, _D), jnp.float32),
        mesh=mesh,
        scratch_types=[
            pltpu.VMEM((_B, _TPW), jnp.int32),
            pltpu.VMEM((2, _TPW, _D), jnp.float32),
            pltpu.VMEM((_NPROMPT, _D), jnp.float32),
            pltpu.SemaphoreType.DMA,
            pltpu.SemaphoreType.DMA,
        ],
    )
    return run(idx.astype(jnp.int32), wte, wpe, reasoning_prompts)


# speedup vs baseline: 1.3950x; 1.1273x over previous
import functools

import jax
import jax.numpy as jnp
from jax import lax
from jax.experimental import pallas as pl
from jax.experimental.pallas import tpu as pltpu
from jax.experimental.pallas import tpu_sc as plsc

_B = 4
_T = 2048
_D = 128
_NPROMPT = 8

_INFO = plsc.get_sparse_core_info()
_NC = _INFO.num_cores
_NS = _INFO.num_subcores
_NW = _NC * _NS
_TPW = _T // _NW
_LANES = 16
_CHUNKS = _D // _LANES


def _body(idx_hbm, wte_hbm, wpe_hbm, prm_hbm, out_hbm,
          idx_v, rows_v, wpe_v, prm_v,
          sem_in, sem_g0, sem_g1, sem_g2, sem_g3, sem_st):
    gsems = (sem_g0, sem_g1, sem_g2, sem_g3)
    wid = lax.axis_index("s") * _NC + lax.axis_index("c")
    t0 = wid * _TPW

    stage = [pltpu.async_copy(idx_hbm.at[b, pl.ds(t0, _TPW)], idx_v.at[b],
                              sem_in) for b in range(_B)]
    stage.append(pltpu.async_copy(wpe_hbm.at[pl.ds(t0, _TPW)], wpe_v, sem_in))

    @pl.when(wid < _B)
    def _():
        pltpu.async_copy(prm_hbm, prm_v, sem_in).wait()
        pltpu.async_copy(prm_v, out_hbm.at[wid, pl.ds(0, _NPROMPT)],
                         sem_st)

    for cp in stage:
        cp.wait()

    gathers = [pltpu.async_copy(wte_hbm.at[idx_v.at[b]], rows_v.at[b],
                                gsems[b]) for b in range(_B)]

    stores = []
    for b in range(_B):
        gathers[b].wait()
        buf = rows_v.at[b]

        def _add_row(i, _):
            for j in range(_CHUNKS):
                sl = pl.ds(j * _LANES, _LANES)
                buf[i, sl] = buf[i, sl] + wpe_v[i, sl]
            return _

        lax.fori_loop(0, _TPW, _add_row, None)
        stores.append(pltpu.async_copy(
            buf, out_hbm.at[b, pl.ds(_NPROMPT + t0, _TPW)], sem_st))

    for cp in stores:
        cp.wait()

    @pl.when(wid < _B)
    def _():
        pltpu.make_async_copy(prm_v, out_hbm.at[wid, pl.ds(0, _NPROMPT)],
                              sem_st).wait()


@functools.partial(jax.jit, static_argnames=())
def kernel(idx, wte, wpe, reasoning_prompts):
    b, t = idx.shape
    assert (b, t) == (_B, _T)
    mesh = plsc.VectorSubcoreMesh(core_axis_name="c", subcore_axis_name="s")
    run = pl.kernel(
        _body,
        out_type=jax.ShapeDtypeStruct((_B, _NPROMPT + _T, _D), jnp.float32),
        mesh=mesh,
        scratch_types=[
            pltpu.VMEM((_B, _TPW), jnp.int32),
            pltpu.VMEM((_B, _TPW, _D), jnp.float32),
            pltpu.VMEM((_TPW, _D), jnp.float32),
            pltpu.VMEM((_NPROMPT, _D), jnp.float32),
            pltpu.SemaphoreType.DMA,
            pltpu.SemaphoreType.DMA,
            pltpu.SemaphoreType.DMA,
            pltpu.SemaphoreType.DMA,
            pltpu.SemaphoreType.DMA,
            pltpu.SemaphoreType.DMA,
        ],
    )
    return run(idx.astype(jnp.int32), wte, wpe, reasoning_prompts)


# speedup vs baseline: 1.3985x; 1.0025x over previous
import functools

import jax
import jax.numpy as jnp
from jax import lax
from jax.experimental import pallas as pl
from jax.experimental.pallas import tpu as pltpu
from jax.experimental.pallas import tpu_sc as plsc

_B = 4
_T = 2048
_D = 128
_NPROMPT = 8

_INFO = plsc.get_sparse_core_info()
_NC = _INFO.num_cores
_NS = _INFO.num_subcores
_NW = _NC * _NS
_TPW = _T // _NW
_LANES = 16
_CHUNKS = _D // _LANES


def _body(idx_hbm, wte_hbm, wpe_hbm, prm_hbm, out_hbm,
          idx_v, rows_v, wpe_v, prm_v,
          sem_in, sem_g0, sem_g1, sem_g2, sem_g3, sem_st):
    gsems = (sem_g0, sem_g1, sem_g2, sem_g3)
    wid = lax.axis_index("s") * _NC + lax.axis_index("c")
    t0 = wid * _TPW

    stage = [pltpu.async_copy(idx_hbm.at[b, pl.ds(t0, _TPW)], idx_v.at[b],
                              sem_in) for b in range(_B)]
    stage.append(pltpu.async_copy(wpe_hbm.at[pl.ds(t0, _TPW)], wpe_v, sem_in))

    @pl.when(wid < _B)
    def _():
        pltpu.async_copy(prm_hbm, prm_v, sem_g0).wait()
        pltpu.async_copy(prm_v, out_hbm.at[wid, pl.ds(0, _NPROMPT)],
                         sem_st)

    for cp in stage:
        cp.wait()

    gathers = [pltpu.async_copy(wte_hbm.at[idx_v.at[b]], rows_v.at[b],
                                gsems[b]) for b in range(_B)]

    stores = []
    for b in range(_B):
        gathers[b].wait()
        buf = rows_v.at[b]

        def _add_row(i, _):
            for j in range(_CHUNKS):
                sl = pl.ds(j * _LANES, _LANES)
                buf[i, sl] = buf[i, sl] + wpe_v[i, sl]
            return _

        lax.fori_loop(0, _TPW, _add_row, None)
        stores.append(pltpu.async_copy(
            buf, out_hbm.at[b, pl.ds(_NPROMPT + t0, _TPW)], sem_st))

    for cp in stores:
        cp.wait()

    @pl.when(wid < _B)
    def _():
        pltpu.make_async_copy(prm_v, out_hbm.at[wid, pl.ds(0, _NPROMPT)],
                              sem_st).wait()


@functools.partial(jax.jit, static_argnames=())
def kernel(idx, wte, wpe, reasoning_prompts):
    b, t = idx.shape
    assert (b, t) == (_B, _T)
    mesh = plsc.VectorSubcoreMesh(core_axis_name="c", subcore_axis_name="s")
    run = pl.kernel(
        _body,
        out_type=jax.ShapeDtypeStruct((_B, _NPROMPT + _T, _D), jnp.float32),
        mesh=mesh,
        scratch_types=[
            pltpu.VMEM((_B, _TPW), jnp.int32),
            pltpu.VMEM((_B, _TPW, _D), jnp.float32),
            pltpu.VMEM((_TPW, _D), jnp.float32),
            pltpu.VMEM((_NPROMPT, _D), jnp.float32),
            pltpu.SemaphoreType.DMA,
            pltpu.SemaphoreType.DMA,
            pltpu.SemaphoreType.DMA,
            pltpu.SemaphoreType.DMA,
            pltpu.SemaphoreType.DMA,
            pltpu.SemaphoreType.DMA,
        ],
    )
    return run(idx.astype(jnp.int32), wte, wpe, reasoning_prompts)
